# Initial kernel scaffold; baseline (speedup 1.0000x reference)
import jax
import jax.numpy as jnp
from jax.experimental import pallas as pl


def kernel(points, nodes, feats, W_enc, W_dec):
    raise NotImplementedError("write your pallas kernel here")


# speedup vs baseline: 2.1126x; 2.1126x over previous
import functools

import jax
import jax.numpy as jnp
from jax import lax
from jax.experimental import pallas as pl
from jax.experimental.pallas import tpu as pltpu
from jax.experimental.pallas import tpu_sc as plsc

N = 50000
M = 4096
K = 32
D = 256

NPAD = 50176
NW = 32
NODES_PER_W = M // NW
PTS_PER_W = NPAD // NW
DEC_CHUNK = 56
STEPS = N // 16

_SC_PARAMS = pltpu.CompilerParams(needs_layout_passes=False)


def _sc_mesh():
    return plsc.VectorSubcoreMesh(core_axis_name="c", subcore_axis_name="s")


def _knn_body(p_ref, nt_ref, out_ref):
    p = p_ref[...]
    nt = nt_ref[...]
    pn = jax.lax.dot_general(p.astype(jnp.bfloat16), nt.astype(jnp.bfloat16),
                             (((1,), (0,)), ((), ())),
                             preferred_element_type=jnp.float32)
    pp = jnp.sum(p * p, axis=1, keepdims=True)
    nn = jnp.sum(nt * nt, axis=0)[None, :]
    d2 = (pp - 2.0 * pn) + nn
    accv = jnp.full((d2.shape[0],), jnp.inf, jnp.float32)
    acci = jnp.zeros((d2.shape[0],), jnp.int32)
    for t in range(M // 1024):
        blk = d2[:, t * 1024:(t + 1) * 1024]
        i_t = jnp.argmin(blk, axis=1).astype(jnp.int32) + t * 1024
        v_t = jnp.min(blk, axis=1)
        upd = (v_t < accv) | ((v_t == accv) & (i_t < acci))
        accv = jnp.where(upd, v_t, accv)
        acci = jnp.where(upd, i_t, acci)
        accv = accv.astype(jnp.bfloat16).astype(jnp.float32)
    out_ref[...] = acci[None, None, :]


def _knn(points_pad, nodes_t):
    grid = NPAD // 512
    out = pl.pallas_call(
        _knn_body,
        grid=(grid,),
        in_specs=[
            pl.BlockSpec((512, 4), lambda i: (i, 0)),
            pl.BlockSpec((4, M), lambda i: (0, 0)),
        ],
        out_specs=pl.BlockSpec((1, 1, 512), lambda i: (i, 0, 0)),
        out_shape=jax.ShapeDtypeStruct((grid, 1, 512), jnp.int32),
    )(points_pad, nodes_t)
    return out.reshape(NPAD)


def _encmm_body(f_ref, we_ref, w1_ref, g_ref, f1_ref):
    f = f_ref[...]
    g_ref[...] = jnp.dot(f, we_ref[...], preferred_element_type=jnp.float32)
    f1_ref[...] = jnp.dot(f, w1_ref[...], preferred_element_type=jnp.float32)


def _encmm(feats_pad, W_enc, W1):
    grid = NPAD // 512
    return pl.pallas_call(
        _encmm_body,
        grid=(grid,),
        in_specs=[
            pl.BlockSpec((512, D), lambda i: (i, 0)),
            pl.BlockSpec((D, D), lambda i: (0, 0)),
            pl.BlockSpec((D, D), lambda i: (0, 0)),
        ],
        out_specs=[
            pl.BlockSpec((512, D), lambda i: (i, 0)),
            pl.BlockSpec((512, D), lambda i: (i, 0)),
        ],
        out_shape=[
            jax.ShapeDtypeStruct((NPAD, D), jnp.float32),
            jax.ShapeDtypeStruct((NPAD, D), jnp.float32),
        ],
    )(feats_pad, W_enc, W1)


def _hmm_body(fc_ref, w2_ref, h_ref):
    h_ref[...] = jnp.dot(fc_ref[...], w2_ref[...],
                         preferred_element_type=jnp.float32)


def _hmm(feats_c, W2):
    return pl.pallas_call(
        _hmm_body,
        grid=(M // 512,),
        in_specs=[
            pl.BlockSpec((512, D), lambda i: (i, 0)),
            pl.BlockSpec((D, D), lambda i: (0, 0)),
        ],
        out_specs=pl.BlockSpec((512, D), lambda i: (i, 0)),
        out_shape=jax.ShapeDtypeStruct((M, D), jnp.float32),
    )(feats_c, W2)


def _grouping(pcd_id):
    @functools.partial(
        pl.kernel,
        mesh=_sc_mesh(),
        out_type=[
            jax.ShapeDtypeStruct((M * K,), jnp.int32),
            jax.ShapeDtypeStruct((M,), jnp.int32),
        ],
        scratch_types=[
            pltpu.VMEM((N,), jnp.int32),
            pltpu.VMEM((NODES_PER_W * K,), jnp.int32),
            pltpu.VMEM((NODES_PER_W,), jnp.int32),
        ],
        compiler_params=_SC_PARAMS,
    )
    def kern(ids_hbm, patch_hbm, counts_hbm, ids_v, patch_v, counts_v):
        wid = lax.axis_index("c") * 16 + lax.axis_index("s")
        lo = wid * NODES_PER_W
        pltpu.sync_copy(ids_hbm, ids_v)

        zeros16 = jnp.zeros((16,), jnp.int32)
        for z in range(NODES_PER_W // 16):
            counts_v[pl.ds(z * 16, 16)] = zeros16

        iota = lax.iota(jnp.int32, 16)
        ones16 = jnp.ones((16,), jnp.int32)

        def step(i, _):
            v = ids_v[pl.ds(i * 16, 16)]
            mine = (v >= lo) & (v < lo + NODES_PER_W)
            lid = jnp.where(mine, v - lo, 0)
            occ, _last = plsc.scan_count(lid, mask=mine)
            prior = plsc.load_gather(counts_v, [lid], mask=mine)
            slot = prior + occ - 1
            valid = mine & (slot < K)
            pidx = i * 16 + iota
            plsc.store_scatter(patch_v, [lid * K + slot], pidx, mask=valid)
            plsc.addupdate_scatter(counts_v, [lid], ones16, mask=mine)
            return 0

        lax.fori_loop(0, STEPS, step, 0)

        def fix(gi, _):
            lids = gi * 16 + iota
            cnt = counts_v[pl.ds(gi * 16, 16)]
            first = plsc.load_gather(patch_v, [lids * K], mask=cnt > 0)
            val = jnp.where(cnt > 0, first, 0)
            for j in range(K):
                plsc.store_scatter(patch_v, [lids * K + j], val,
                                   mask=cnt <= j)
            return 0

        lax.fori_loop(0, NODES_PER_W // 16, fix, 0)

        pltpu.sync_copy(patch_v, patch_hbm.at[pl.ds(lo * K, NODES_PER_W * K)])
        pltpu.sync_copy(counts_v, counts_hbm.at[pl.ds(lo, NODES_PER_W)])

    return kern(pcd_id)


def _gathermax(g, patch, counts):
    @functools.partial(
        pl.kernel,
        mesh=_sc_mesh(),
        out_type=jax.ShapeDtypeStruct((M, D), jnp.float32),
        scratch_types=[
            pltpu.VMEM((NODES_PER_W * K,), jnp.int32),
            pltpu.VMEM((NODES_PER_W + 16,), jnp.int32),
            pltpu.VMEM((D,), jnp.float32),
            pltpu.VMEM((K, D), jnp.float32),
            pltpu.VMEM((K, D), jnp.float32),
            pltpu.VMEM((NODES_PER_W, D), jnp.float32),
            pltpu.SemaphoreType.DMA,
            pltpu.SemaphoreType.DMA,
        ],
        compiler_params=_SC_PARAMS,
    )
    def kern(g_hbm, patch_hbm, counts_hbm, out_hbm,
             patch_v, cnt_v, g0_v, rows0, rows1, out_v, sem0, sem1):
        wid = lax.axis_index("c") * 16 + lax.axis_index("s")
        lo = wid * NODES_PER_W
        pltpu.sync_copy(patch_hbm.at[pl.ds(lo * K, NODES_PER_W * K)], patch_v)
        pltpu.sync_copy(counts_hbm.at[pl.ds(lo, NODES_PER_W)],
                        cnt_v.at[pl.ds(0, NODES_PER_W)])
        pltpu.sync_copy(g_hbm.at[0], g0_v)

        def fire(n, buf, sem):
            pltpu.async_copy(g_hbm.at[patch_v.at[pl.ds(n * K, K)]],
                             buf, sem)

        fire(0, rows0, sem0)

        def loop_body(n, _):
            is_even = lax.rem(n, 2) == 0

            @pl.when((n + 1 < NODES_PER_W) & is_even)
            def _():
                fire(n + 1, rows1, sem1)

            @pl.when((n + 1 < NODES_PER_W) & jnp.logical_not(is_even))
            def _():
                fire(n + 1, rows0, sem0)

            cnt = cnt_v[pl.ds(n, 16)][0]
            pad = cnt < K

            def reduce_from(buf, sem):
                pltpu.make_async_copy(g_hbm.at[patch_v.at[pl.ds(0, K)]],
                                      buf, sem).wait()
                for c in range(D // 16):
                    s = pl.ds(c * 16, 16)
                    acc = buf[0, s]
                    for j in range(1, K):
                        acc = jnp.maximum(acc, buf[j, s])
                    acc = jnp.where(pad, jnp.maximum(acc, g0_v[s]), acc)
                    out_v[n, s] = jnp.maximum(acc, 0.0)

            @pl.when(is_even)
            def _():
                reduce_from(rows0, sem0)

            @pl.when(jnp.logical_not(is_even))
            def _():
                reduce_from(rows1, sem1)

            return 0

        lax.fori_loop(0, NODES_PER_W, loop_body, 0)
        pltpu.sync_copy(out_v, out_hbm.at[pl.ds(lo, NODES_PER_W)])

    return kern(g, patch, counts)


def _decfuse(h, pcd_pad, f1):
    n_chunks = PTS_PER_W // DEC_CHUNK

    @functools.partial(
        pl.kernel,
        mesh=_sc_mesh(),
        out_type=jax.ShapeDtypeStruct((NPAD, D), jnp.float32),
        scratch_types=[
            pltpu.VMEM((PTS_PER_W,), jnp.int32),
            pltpu.VMEM((DEC_CHUNK, D), jnp.float32),
            pltpu.VMEM((DEC_CHUNK, D), jnp.float32),
            pltpu.VMEM((DEC_CHUNK, D), jnp.float32),
            pltpu.VMEM((DEC_CHUNK, D), jnp.float32),
            pltpu.SemaphoreType.DMA,
            pltpu.SemaphoreType.DMA,
            pltpu.SemaphoreType.DMA,
            pltpu.SemaphoreType.DMA,
        ],
        compiler_params=_SC_PARAMS,
    )
    def kern(h_hbm, pcd_hbm, f1_hbm, out_hbm,
             idx_v, h0, h1, fb0, fb1, sh0, sh1, sf0, sf1):
        wid = lax.axis_index("c") * 16 + lax.axis_index("s")
        base = wid * PTS_PER_W
        pltpu.sync_copy(pcd_hbm.at[pl.ds(base, PTS_PER_W)], idx_v)

        hb = (h0, h1)
        fb = (fb0, fb1)
        shs = (sh0, sh1)
        sfs = (sf0, sf1)

        def fire(ci, b):
            off = ci * DEC_CHUNK
            pltpu.async_copy(h_hbm.at[idx_v.at[pl.ds(off, DEC_CHUNK)]],
                             hb[b], shs[b])
            pltpu.async_copy(f1_hbm.at[pl.ds(base + off, DEC_CHUNK)],
                             fb[b], sfs[b])

        fire(0, 0)

        def chunk(ci, _):
            is_even = lax.rem(ci, 2) == 0

            @pl.when((ci + 1 < n_chunks) & is_even)
            def _():
                fire(ci + 1, 1)

            @pl.when((ci + 1 < n_chunks) & jnp.logical_not(is_even))
            def _():
                fire(ci + 1, 0)

            def work(b):
                pltpu.make_async_copy(
                    h_hbm.at[idx_v.at[pl.ds(0, DEC_CHUNK)]],
                    hb[b], shs[b]).wait()
                pltpu.make_async_copy(
                    f1_hbm.at[pl.ds(base, DEC_CHUNK)],
                    fb[b], sfs[b]).wait()

                def row(r, _):
                    for c in range(D // 16):
                        s = pl.ds(c * 16, 16)
                        fb[b][r, s] = jnp.maximum(
                            hb[b][r, s] + fb[b][r, s], 0.0)
                    return 0

                lax.fori_loop(0, DEC_CHUNK, row, 0)
                pltpu.sync_copy(
                    fb[b], out_hbm.at[pl.ds(base + ci * DEC_CHUNK,
                                            DEC_CHUNK)])

            @pl.when(is_even)
            def _():
                work(0)

            @pl.when(jnp.logical_not(is_even))
            def _():
                work(1)

            return 0

        lax.fori_loop(0, n_chunks, chunk, 0)

    return kern(h, pcd_pad, f1)


def kernel(points, nodes, feats, W_enc, W_dec):
    points_pad = jnp.pad(points, ((0, NPAD - N), (0, 1)))
    nodes_t = jnp.pad(nodes, ((0, 0), (0, 1))).T
    feats_pad = jnp.pad(feats, ((0, NPAD - N), (0, 0)))
    W1 = W_dec[:D]
    W2 = W_dec[D:]

    pcd_full = _knn(points_pad, nodes_t)
    pcd = pcd_full[:N]
    pcd_pad = jnp.pad(pcd, (0, NPAD - N))

    patch, counts = _grouping(pcd)
    g, f1 = _encmm(feats_pad, W_enc, W1)
    feats_c = _gathermax(g, patch, counts)
    h = _hmm(feats_c, W2)
    ff = _decfuse(h, pcd_pad, f1)
    return (feats_c, ff[:N])
